# baseline (speedup 1.0000x reference)
import functools

import jax
import jax.numpy as jnp
from jax.experimental import pallas as pl

_C = 12
_GD = 8
_GH = 16
_GW = 16
_CI = 3
_CO = 3
_NIN = 4
_TH = 16


def _upsample_body(grid_ref, out_ref, *, W):
    sx = W // _GW
    xi = jax.lax.broadcasted_iota(jnp.int32, (_GW, W), 0).astype(jnp.float32)
    wi = jax.lax.broadcasted_iota(jnp.int32, (_GW, W), 1).astype(jnp.float32)
    ux = jnp.clip((wi + 0.5) / sx - 0.5, 0.0, _GW - 1.0)
    mxT = jnp.maximum(1.0 - jnp.abs(xi - ux), 0.0)
    g = grid_ref[0].reshape(_GH * _C * _GD, _GW)
    out_ref[0] = jnp.dot(
        g, mxT, preferred_element_type=jnp.float32
    ).reshape(_GH, _C * _GD, W)


def _slice_body(rowa_ref, rowb_ref, guide_ref, inp_ref, out_ref, *, H, W):
    t = pl.program_id(1)
    sy = H // _GH

    hh = (
        jax.lax.broadcasted_iota(jnp.int32, (_TH, 1), 0).astype(jnp.float32)
        + t * _TH
    )
    uy = jnp.clip((hh + 0.5) / sy - 0.5, 0.0, _GH - 1.0)
    S = jnp.clip((t - 1) // 2, 0, _GH - 2)
    w1 = jnp.clip(uy - S.astype(jnp.float32), 0.0, 1.0)
    w0 = 1.0 - w1

    HW = W // 2
    for h0 in (0, HW):
        sl = slice(h0, h0 + HW)
        uz = jnp.clip(
            guide_ref[0, :, sl] * _GD - 0.5, 0.0, _GD - 1.0
        )
        inp = [inp_ref[0, i, :, sl] for i in range(_CI)]
        acc = [None] * _CO
        for k in range(_GD):
            wk = jnp.maximum(1.0 - jnp.abs(uz - float(k)), 0.0)
            wa = wk * w0
            wb = wk * w1
            for c in range(_CO):
                base = (_NIN * c) * _GD + k
                ta = rowa_ref[0, 0, base + _CI * _GD, sl]
                tb = rowb_ref[0, 0, base + _CI * _GD, sl]
                for i in range(_CI):
                    ta = ta + rowa_ref[0, 0, base + i * _GD, sl] * inp[i]
                    tb = tb + rowb_ref[0, 0, base + i * _GD, sl] * inp[i]
                term = wa * ta + wb * tb
                acc[c] = term if acc[c] is None else acc[c] + term
        for c in range(_CO):
            out_ref[0, c, :, sl] = acc[c]


def _row_idx(t):
    return jnp.clip((t - 1) // 2, 0, _GH - 2)


def kernel(bilateral_grid, guide, input):
    N, C, gd, gh, gw = bilateral_grid.shape
    _, ci, H, W = input.shape
    grid_r = jnp.transpose(bilateral_grid, (0, 3, 1, 2, 4)).reshape(
        N, gh, C * gd, gw
    )
    gx = pl.pallas_call(
        functools.partial(_upsample_body, W=W),
        grid=(N,),
        in_specs=[pl.BlockSpec((1, gh, C * gd, gw), lambda n: (n, 0, 0, 0))],
        out_specs=pl.BlockSpec((1, gh, C * gd, W), lambda n: (n, 0, 0, 0)),
        out_shape=jax.ShapeDtypeStruct((N, gh, C * gd, W), jnp.float32),
    )(grid_r)

    nt = H // _TH
    body = functools.partial(_slice_body, H=H, W=W)
    return pl.pallas_call(
        body,
        grid=(N, nt),
        in_specs=[
            pl.BlockSpec(
                (1, 1, C * gd, W), lambda n, t: (n, _row_idx(t), 0, 0)
            ),
            pl.BlockSpec(
                (1, 1, C * gd, W), lambda n, t: (n, _row_idx(t) + 1, 0, 0)
            ),
            pl.BlockSpec((1, _TH, W), lambda n, t: (n, t, 0)),
            pl.BlockSpec((1, ci, _TH, W), lambda n, t: (n, 0, t, 0)),
        ],
        out_specs=pl.BlockSpec((1, _CO, _TH, W), lambda n, t: (n, 0, t, 0)),
        out_shape=jax.ShapeDtypeStruct((N, _CO, H, W), jnp.float32),
    )(gx, gx, guide, input)


# speedup vs baseline: 1.1334x; 1.1334x over previous
import functools

import jax
import jax.numpy as jnp
from jax.experimental import pallas as pl
from jax.experimental.pallas import tpu as pltpu

_C = 12
_GD = 8
_GH = 16
_GW = 16
_CI = 3
_CO = 3
_NIN = 4
_TH = 16


def _slice_body(grid_ref, guide_ref, inp_ref, out_ref, gx_s, *, H, W):
    t = pl.program_id(1)
    sx = W // _GW
    sy = H // _GH

    @pl.when(t == 0)
    def _():
        xi = jax.lax.broadcasted_iota(jnp.int32, (_GW, W), 0).astype(
            jnp.float32
        )
        wi = jax.lax.broadcasted_iota(jnp.int32, (_GW, W), 1).astype(
            jnp.float32
        )
        ux = jnp.clip((wi + 0.5) / sx - 0.5, 0.0, _GW - 1.0)
        mxT = jnp.maximum(1.0 - jnp.abs(xi - ux), 0.0)
        g = grid_ref[0].reshape(_GH * _C * _GD, _GW)
        gx_s[...] = jnp.dot(
            g, mxT, preferred_element_type=jnp.float32
        ).reshape(_GH, _C * _GD, W)

    hh = (
        jax.lax.broadcasted_iota(jnp.int32, (_TH, 1), 0).astype(jnp.float32)
        + t * _TH
    )
    uy = jnp.clip((hh + 0.5) / sy - 0.5, 0.0, _GH - 1.0)
    S = jnp.clip((t - 1) // 2, 0, _GH - 2)
    w1 = jnp.clip(uy - S.astype(jnp.float32), 0.0, 1.0)
    w0 = 1.0 - w1

    HW = W // 2
    for h0 in (0, HW):
        sl = slice(h0, h0 + HW)
        uz = jnp.clip(
            guide_ref[0, :, sl] * _GD - 0.5, 0.0, _GD - 1.0
        )
        inp = [inp_ref[0, i, :, sl] for i in range(_CI)]
        acc = [None] * _CO
        for k in range(_GD):
            wk = jnp.maximum(1.0 - jnp.abs(uz - float(k)), 0.0)
            wa = wk * w0
            wb = wk * w1
            for c in range(_CO):
                base = (_NIN * c) * _GD + k
                ta = gx_s[S, base + _CI * _GD, sl]
                tb = gx_s[S + 1, base + _CI * _GD, sl]
                for i in range(_CI):
                    ta = ta + gx_s[S, base + i * _GD, sl] * inp[i]
                    tb = tb + gx_s[S + 1, base + i * _GD, sl] * inp[i]
                term = wa * ta + wb * tb
                acc[c] = term if acc[c] is None else acc[c] + term
        for c in range(_CO):
            out_ref[0, c, :, sl] = acc[c]


def kernel(bilateral_grid, guide, input):
    N, C, gd, gh, gw = bilateral_grid.shape
    _, ci, H, W = input.shape
    grid_r = jnp.transpose(bilateral_grid, (0, 3, 1, 2, 4)).reshape(
        N, gh, C * gd, gw
    )
    nt = H // _TH
    body = functools.partial(_slice_body, H=H, W=W)
    return pl.pallas_call(
        body,
        grid=(N, nt),
        in_specs=[
            pl.BlockSpec((1, gh, C * gd, gw), lambda n, t: (n, 0, 0, 0)),
            pl.BlockSpec((1, _TH, W), lambda n, t: (n, t, 0)),
            pl.BlockSpec((1, ci, _TH, W), lambda n, t: (n, 0, t, 0)),
        ],
        out_specs=pl.BlockSpec((1, _CO, _TH, W), lambda n, t: (n, 0, t, 0)),
        out_shape=jax.ShapeDtypeStruct((N, _CO, H, W), jnp.float32),
        scratch_shapes=[pltpu.VMEM((gh, C * gd, W), jnp.float32)],
        compiler_params=pltpu.CompilerParams(
            dimension_semantics=("parallel", "arbitrary"),
        ),
    )(grid_r, guide, input)


# speedup vs baseline: 1.5431x; 1.3615x over previous
import functools

import jax
import jax.numpy as jnp
from jax.experimental import pallas as pl
from jax.experimental.pallas import tpu as pltpu

_C = 12
_GD = 8
_GH = 16
_GW = 16
_CI = 3
_CO = 3
_NIN = 4
_SUB = 16
_NSUB = 2
_TH = _SUB * _NSUB


def _slice_body(grid_ref, guide_ref, inp_ref, out_ref, gx_s, *, H, W):
    t = pl.program_id(1)
    sx = W // _GW
    sy = H // _GH

    @pl.when(t == 0)
    def _():
        xi = jax.lax.broadcasted_iota(jnp.int32, (_GW, W), 0).astype(
            jnp.float32
        )
        wi = jax.lax.broadcasted_iota(jnp.int32, (_GW, W), 1).astype(
            jnp.float32
        )
        ux = jnp.clip((wi + 0.5) / sx - 0.5, 0.0, _GW - 1.0)
        mxT = jnp.maximum(1.0 - jnp.abs(xi - ux), 0.0)
        g = grid_ref[0].reshape(_GH * _C * _GD, _GW)
        gx_s[...] = jnp.dot(
            g, mxT, preferred_element_type=jnp.float32
        ).reshape(_GH, _C * _GD, W)

    HW = W // 2
    for s in range(_NSUB):
        ts = t * _NSUB + s
        r0 = s * _SUB
        hh = (
            jax.lax.broadcasted_iota(jnp.int32, (_SUB, 1), 0).astype(
                jnp.float32
            )
            + ts * _SUB
        )
        uy = jnp.clip((hh + 0.5) / sy - 0.5, 0.0, _GH - 1.0)
        S = jnp.clip((ts - 1) // 2, 0, _GH - 2)
        w1 = jnp.clip(uy - S.astype(jnp.float32), 0.0, 1.0)
        w0 = 1.0 - w1
        rows = slice(r0, r0 + _SUB)

        for h0 in (0, HW):
            sl = slice(h0, h0 + HW)
            uz = jnp.clip(
                guide_ref[0, rows, sl] * _GD - 0.5, 0.0, _GD - 1.0
            )
            inp = [inp_ref[0, i, rows, sl] for i in range(_CI)]
            acc = [None] * _CO
            for k in range(_GD):
                wk = jnp.maximum(1.0 - jnp.abs(uz - float(k)), 0.0)
                wa = wk * w0
                wb = wk * w1
                for c in range(_CO):
                    base = (_NIN * c) * _GD + k
                    ta = gx_s[S, base + _CI * _GD, sl]
                    tb = gx_s[S + 1, base + _CI * _GD, sl]
                    for i in range(_CI):
                        ta = ta + gx_s[S, base + i * _GD, sl] * inp[i]
                        tb = tb + gx_s[S + 1, base + i * _GD, sl] * inp[i]
                    term = wa * ta + wb * tb
                    acc[c] = term if acc[c] is None else acc[c] + term
            for c in range(_CO):
                out_ref[0, c, rows, sl] = acc[c]


def kernel(bilateral_grid, guide, input):
    N, C, gd, gh, gw = bilateral_grid.shape
    _, ci, H, W = input.shape
    grid_r = jnp.transpose(bilateral_grid, (0, 3, 1, 2, 4)).reshape(
        N, gh, C * gd, gw
    )
    nt = H // _TH
    body = functools.partial(_slice_body, H=H, W=W)
    return pl.pallas_call(
        body,
        grid=(N, nt),
        in_specs=[
            pl.BlockSpec((1, gh, C * gd, gw), lambda n, t: (n, 0, 0, 0)),
            pl.BlockSpec((1, _TH, W), lambda n, t: (n, t, 0)),
            pl.BlockSpec((1, ci, _TH, W), lambda n, t: (n, 0, t, 0)),
        ],
        out_specs=pl.BlockSpec((1, _CO, _TH, W), lambda n, t: (n, 0, t, 0)),
        out_shape=jax.ShapeDtypeStruct((N, _CO, H, W), jnp.float32),
        scratch_shapes=[pltpu.VMEM((gh, C * gd, W), jnp.float32)],
        compiler_params=pltpu.CompilerParams(
            dimension_semantics=("parallel", "arbitrary"),
        ),
    )(grid_r, guide, input)


# speedup vs baseline: 1.6186x; 1.0489x over previous
import functools

import jax
import jax.numpy as jnp
from jax.experimental import pallas as pl
from jax.experimental.pallas import tpu as pltpu

_C = 12
_GD = 8
_GH = 16
_GW = 16
_CI = 3
_CO = 3
_NIN = 4
_SUB = 16
_NSUB = 4
_TH = _SUB * _NSUB


def _slice_body(grid_ref, guide_ref, inp_ref, out_ref, gx_s, *, H, W):
    t = pl.program_id(1)
    sx = W // _GW
    sy = H // _GH

    @pl.when(t == 0)
    def _():
        xi = jax.lax.broadcasted_iota(jnp.int32, (_GW, W), 0).astype(
            jnp.float32
        )
        wi = jax.lax.broadcasted_iota(jnp.int32, (_GW, W), 1).astype(
            jnp.float32
        )
        ux = jnp.clip((wi + 0.5) / sx - 0.5, 0.0, _GW - 1.0)
        mxT = jnp.maximum(1.0 - jnp.abs(xi - ux), 0.0)
        g = grid_ref[0].reshape(_GH * _C * _GD, _GW)
        gx_s[...] = jnp.dot(
            g, mxT, preferred_element_type=jnp.float32
        ).reshape(_GH, _C * _GD, W)

    HW = W // 2
    for s in range(_NSUB):
        ts = t * _NSUB + s
        r0 = s * _SUB
        hh = (
            jax.lax.broadcasted_iota(jnp.int32, (_SUB, 1), 0).astype(
                jnp.float32
            )
            + ts * _SUB
        )
        uy = jnp.clip((hh + 0.5) / sy - 0.5, 0.0, _GH - 1.0)
        S = jnp.clip((ts - 1) // 2, 0, _GH - 2)
        w1 = jnp.clip(uy - S.astype(jnp.float32), 0.0, 1.0)
        w0 = 1.0 - w1
        rows = slice(r0, r0 + _SUB)

        for h0 in (0, HW):
            sl = slice(h0, h0 + HW)
            uz = jnp.clip(
                guide_ref[0, rows, sl] * _GD - 0.5, 0.0, _GD - 1.0
            )
            inp = [inp_ref[0, i, rows, sl] for i in range(_CI)]
            acc = [None] * _CO
            for k in range(_GD):
                wk = jnp.maximum(1.0 - jnp.abs(uz - float(k)), 0.0)
                wa = wk * w0
                wb = wk * w1
                for c in range(_CO):
                    base = (_NIN * c) * _GD + k
                    ta = gx_s[S, base + _CI * _GD, sl]
                    tb = gx_s[S + 1, base + _CI * _GD, sl]
                    for i in range(_CI):
                        ta = ta + gx_s[S, base + i * _GD, sl] * inp[i]
                        tb = tb + gx_s[S + 1, base + i * _GD, sl] * inp[i]
                    term = wa * ta + wb * tb
                    acc[c] = term if acc[c] is None else acc[c] + term
            for c in range(_CO):
                out_ref[0, c, rows, sl] = acc[c]


def kernel(bilateral_grid, guide, input):
    N, C, gd, gh, gw = bilateral_grid.shape
    _, ci, H, W = input.shape
    grid_r = jnp.transpose(bilateral_grid, (0, 3, 1, 2, 4)).reshape(
        N, gh, C * gd, gw
    )
    nt = H // _TH
    body = functools.partial(_slice_body, H=H, W=W)
    return pl.pallas_call(
        body,
        grid=(N, nt),
        in_specs=[
            pl.BlockSpec((1, gh, C * gd, gw), lambda n, t: (n, 0, 0, 0)),
            pl.BlockSpec((1, _TH, W), lambda n, t: (n, t, 0)),
            pl.BlockSpec((1, ci, _TH, W), lambda n, t: (n, 0, t, 0)),
        ],
        out_specs=pl.BlockSpec((1, _CO, _TH, W), lambda n, t: (n, 0, t, 0)),
        out_shape=jax.ShapeDtypeStruct((N, _CO, H, W), jnp.float32),
        scratch_shapes=[pltpu.VMEM((gh, C * gd, W), jnp.float32)],
        compiler_params=pltpu.CompilerParams(
            dimension_semantics=("parallel", "arbitrary"),
        ),
    )(grid_r, guide, input)


# speedup vs baseline: 1.6393x; 1.0128x over previous
import functools

import jax
import jax.numpy as jnp
from jax.experimental import pallas as pl
from jax.experimental.pallas import tpu as pltpu

_C = 12
_GD = 8
_GH = 16
_GW = 16
_CI = 3
_CO = 3
_NIN = 4
_SUB = 16
_NSUB = 8
_TH = _SUB * _NSUB


def _slice_body(grid_ref, guide_ref, inp_ref, out_ref, gx_s, *, H, W):
    t = pl.program_id(1)
    sx = W // _GW
    sy = H // _GH

    @pl.when(t == 0)
    def _():
        xi = jax.lax.broadcasted_iota(jnp.int32, (_GW, W), 0).astype(
            jnp.float32
        )
        wi = jax.lax.broadcasted_iota(jnp.int32, (_GW, W), 1).astype(
            jnp.float32
        )
        ux = jnp.clip((wi + 0.5) / sx - 0.5, 0.0, _GW - 1.0)
        mxT = jnp.maximum(1.0 - jnp.abs(xi - ux), 0.0)
        g = grid_ref[0].reshape(_GH * _C * _GD, _GW)
        gx_s[...] = jnp.dot(
            g, mxT, preferred_element_type=jnp.float32
        ).reshape(_GH, _C * _GD, W)

    HW = W // 2
    for s in range(_NSUB):
        ts = t * _NSUB + s
        r0 = s * _SUB
        hh = (
            jax.lax.broadcasted_iota(jnp.int32, (_SUB, 1), 0).astype(
                jnp.float32
            )
            + ts * _SUB
        )
        uy = jnp.clip((hh + 0.5) / sy - 0.5, 0.0, _GH - 1.0)
        S = jnp.clip((ts - 1) // 2, 0, _GH - 2)
        w1 = jnp.clip(uy - S.astype(jnp.float32), 0.0, 1.0)
        w0 = 1.0 - w1
        rows = slice(r0, r0 + _SUB)

        for h0 in (0, HW):
            sl = slice(h0, h0 + HW)
            uz = jnp.clip(
                guide_ref[0, rows, sl] * _GD - 0.5, 0.0, _GD - 1.0
            )
            inp = [inp_ref[0, i, rows, sl] for i in range(_CI)]
            acc = [None] * _CO
            for k in range(_GD):
                wk = jnp.maximum(1.0 - jnp.abs(uz - float(k)), 0.0)
                wa = wk * w0
                wb = wk * w1
                for c in range(_CO):
                    base = (_NIN * c) * _GD + k
                    ta = gx_s[S, base + _CI * _GD, sl]
                    tb = gx_s[S + 1, base + _CI * _GD, sl]
                    for i in range(_CI):
                        ta = ta + gx_s[S, base + i * _GD, sl] * inp[i]
                        tb = tb + gx_s[S + 1, base + i * _GD, sl] * inp[i]
                    term = wa * ta + wb * tb
                    acc[c] = term if acc[c] is None else acc[c] + term
            for c in range(_CO):
                out_ref[0, c, rows, sl] = acc[c]


def kernel(bilateral_grid, guide, input):
    N, C, gd, gh, gw = bilateral_grid.shape
    _, ci, H, W = input.shape
    grid_r = jnp.transpose(bilateral_grid, (0, 3, 1, 2, 4)).reshape(
        N, gh, C * gd, gw
    )
    nt = H // _TH
    body = functools.partial(_slice_body, H=H, W=W)
    return pl.pallas_call(
        body,
        grid=(N, nt),
        in_specs=[
            pl.BlockSpec((1, gh, C * gd, gw), lambda n, t: (n, 0, 0, 0)),
            pl.BlockSpec((1, _TH, W), lambda n, t: (n, t, 0)),
            pl.BlockSpec((1, ci, _TH, W), lambda n, t: (n, 0, t, 0)),
        ],
        out_specs=pl.BlockSpec((1, _CO, _TH, W), lambda n, t: (n, 0, t, 0)),
        out_shape=jax.ShapeDtypeStruct((N, _CO, H, W), jnp.float32),
        scratch_shapes=[pltpu.VMEM((gh, C * gd, W), jnp.float32)],
        compiler_params=pltpu.CompilerParams(
            dimension_semantics=("parallel", "arbitrary"),
        ),
    )(grid_r, guide, input)


# speedup vs baseline: 1.6584x; 1.0116x over previous
import functools

import jax
import jax.numpy as jnp
from jax.experimental import pallas as pl
from jax.experimental.pallas import tpu as pltpu

_C = 12
_GD = 8
_GH = 16
_GW = 16
_CI = 3
_CO = 3
_NIN = 4
_SUB = 16
_NSUB = 8
_TH = _SUB * _NSUB


def _slice_body(grid_ref, guide_ref, inp_ref, out_ref, gx_s, *, H, W):
    t = pl.program_id(1)
    sx = W // _GW
    sy = H // _GH

    @pl.when(t == 0)
    def _():
        xi = jax.lax.broadcasted_iota(jnp.int32, (_GW, W), 0).astype(
            jnp.float32
        )
        wi = jax.lax.broadcasted_iota(jnp.int32, (_GW, W), 1).astype(
            jnp.float32
        )
        ux = jnp.clip((wi + 0.5) / sx - 0.5, 0.0, _GW - 1.0)
        mxT = jnp.maximum(1.0 - jnp.abs(xi - ux), 0.0)
        g = grid_ref[0].reshape(_GH * _C * _GD, _GW)
        gx_s[...] = jnp.dot(
            g, mxT, preferred_element_type=jnp.float32
        ).reshape(_GH, _C * _GD, W)

    HW = W
    for s in range(_NSUB):
        ts = t * _NSUB + s
        r0 = s * _SUB
        hh = (
            jax.lax.broadcasted_iota(jnp.int32, (_SUB, 1), 0).astype(
                jnp.float32
            )
            + ts * _SUB
        )
        uy = jnp.clip((hh + 0.5) / sy - 0.5, 0.0, _GH - 1.0)
        S = jnp.clip((ts - 1) // 2, 0, _GH - 2)
        w1 = jnp.clip(uy - S.astype(jnp.float32), 0.0, 1.0)
        w0 = 1.0 - w1
        rows = slice(r0, r0 + _SUB)

        for h0 in (0,):
            sl = slice(h0, h0 + HW)
            uz = jnp.clip(
                guide_ref[0, rows, sl] * _GD - 0.5, 0.0, _GD - 1.0
            )
            inp = [inp_ref[0, i, rows, sl] for i in range(_CI)]
            acc = [None] * _CO
            for k in range(_GD):
                wk = jnp.maximum(1.0 - jnp.abs(uz - float(k)), 0.0)
                wa = wk * w0
                wb = wk * w1
                for c in range(_CO):
                    base = (_NIN * c) * _GD + k
                    ta = gx_s[S, base + _CI * _GD, sl]
                    tb = gx_s[S + 1, base + _CI * _GD, sl]
                    for i in range(_CI):
                        ta = ta + gx_s[S, base + i * _GD, sl] * inp[i]
                        tb = tb + gx_s[S + 1, base + i * _GD, sl] * inp[i]
                    term = wa * ta + wb * tb
                    acc[c] = term if acc[c] is None else acc[c] + term
            for c in range(_CO):
                out_ref[0, c, rows, sl] = acc[c]


def kernel(bilateral_grid, guide, input):
    N, C, gd, gh, gw = bilateral_grid.shape
    _, ci, H, W = input.shape
    grid_r = jnp.transpose(bilateral_grid, (0, 3, 1, 2, 4)).reshape(
        N, gh, C * gd, gw
    )
    nt = H // _TH
    body = functools.partial(_slice_body, H=H, W=W)
    return pl.pallas_call(
        body,
        grid=(N, nt),
        in_specs=[
            pl.BlockSpec((1, gh, C * gd, gw), lambda n, t: (n, 0, 0, 0)),
            pl.BlockSpec((1, _TH, W), lambda n, t: (n, t, 0)),
            pl.BlockSpec((1, ci, _TH, W), lambda n, t: (n, 0, t, 0)),
        ],
        out_specs=pl.BlockSpec((1, _CO, _TH, W), lambda n, t: (n, 0, t, 0)),
        out_shape=jax.ShapeDtypeStruct((N, _CO, H, W), jnp.float32),
        scratch_shapes=[pltpu.VMEM((gh, C * gd, W), jnp.float32)],
        compiler_params=pltpu.CompilerParams(
            dimension_semantics=("parallel", "arbitrary"),
        ),
    )(grid_r, guide, input)


# speedup vs baseline: 1.6586x; 1.0001x over previous
import functools

import jax
import jax.numpy as jnp
from jax.experimental import pallas as pl
from jax.experimental.pallas import tpu as pltpu

_C = 12
_GD = 8
_GH = 16
_GW = 16
_CI = 3
_CO = 3
_NIN = 4
_SUB = 16
_NSUB = 8
_TH = _SUB * _NSUB


def _slice_body(grid_ref, guide_ref, inp_ref, out_ref, gx_s, *, H, W):
    t = pl.program_id(1)
    sx = W // _GW
    sy = H // _GH

    @pl.when(t == 0)
    def _():
        xi = jax.lax.broadcasted_iota(jnp.int32, (_GW, W), 0).astype(
            jnp.float32
        )
        wi = jax.lax.broadcasted_iota(jnp.int32, (_GW, W), 1).astype(
            jnp.float32
        )
        ux = jnp.clip((wi + 0.5) / sx - 0.5, 0.0, _GW - 1.0)
        mxT = jnp.maximum(1.0 - jnp.abs(xi - ux), 0.0)
        g = grid_ref[0].reshape(_GH * _C * _GD, _GW)
        gx_s[...] = jnp.dot(
            g, mxT, preferred_element_type=jnp.float32
        ).reshape(_GH, _C * _GD, W)

    for s in range(_NSUB):
        ts = t * _NSUB + s
        r0 = s * _SUB
        hh = (
            jax.lax.broadcasted_iota(jnp.int32, (_SUB, 1), 0).astype(
                jnp.float32
            )
            + ts * _SUB
        )
        uy = jnp.clip((hh + 0.5) / sy - 0.5, 0.0, _GH - 1.0)
        S = jnp.clip((ts - 1) // 2, 0, _GH - 2)
        w1 = jnp.clip(uy - S.astype(jnp.float32), 0.0, 1.0)
        w0 = 1.0 - w1
        rows = slice(r0, r0 + _SUB)

        uz = jnp.clip(guide_ref[0, rows] * _GD - 0.5, 0.0, _GD - 1.0)
        inp = [inp_ref[0, i, rows] for i in range(_CI)]
        acc = [None] * _CO
        for k in range(_GD):
            wk = jnp.maximum(1.0 - jnp.abs(uz - float(k)), 0.0)
            wa = wk * w0
            wb = wk * w1
            for c in range(_CO):
                base = (_NIN * c) * _GD + k
                ta = gx_s[S, base + _CI * _GD]
                tb = gx_s[S + 1, base + _CI * _GD]
                for i in range(_CI):
                    ta = ta + gx_s[S, base + i * _GD] * inp[i]
                    tb = tb + gx_s[S + 1, base + i * _GD] * inp[i]
                term = wa * ta + wb * tb
                acc[c] = term if acc[c] is None else acc[c] + term
        for c in range(_CO):
            out_ref[0, c, rows] = acc[c]


def kernel(bilateral_grid, guide, input):
    N, C, gd, gh, gw = bilateral_grid.shape
    _, ci, H, W = input.shape
    grid_r = jnp.transpose(bilateral_grid, (0, 3, 1, 2, 4)).reshape(
        N, gh, C * gd, gw
    )
    nt = H // _TH
    body = functools.partial(_slice_body, H=H, W=W)
    return pl.pallas_call(
        body,
        grid=(N, nt),
        in_specs=[
            pl.BlockSpec((1, gh, C * gd, gw), lambda n, t: (n, 0, 0, 0)),
            pl.BlockSpec((1, _TH, W), lambda n, t: (n, t, 0)),
            pl.BlockSpec((1, ci, _TH, W), lambda n, t: (n, 0, t, 0)),
        ],
        out_specs=pl.BlockSpec((1, _CO, _TH, W), lambda n, t: (n, 0, t, 0)),
        out_shape=jax.ShapeDtypeStruct((N, _CO, H, W), jnp.float32),
        scratch_shapes=[pltpu.VMEM((gh, C * gd, W), jnp.float32)],
        compiler_params=pltpu.CompilerParams(
            dimension_semantics=("parallel", "arbitrary"),
        ),
    )(grid_r, guide, input)


# speedup vs baseline: 1.6591x; 1.0003x over previous
import functools

import jax
import jax.numpy as jnp
from jax.experimental import pallas as pl
from jax.experimental.pallas import tpu as pltpu

_C = 12
_GD = 8
_GH = 16
_GW = 16
_CI = 3
_CO = 3
_NIN = 4
_SUB = 16
_NSUB = 16
_TH = _SUB * _NSUB


def _slice_body(grid_ref, guide_ref, inp_ref, out_ref, gx_s, *, H, W):
    t = pl.program_id(1)
    sx = W // _GW
    sy = H // _GH

    @pl.when(t == 0)
    def _():
        xi = jax.lax.broadcasted_iota(jnp.int32, (_GW, W), 0).astype(
            jnp.float32
        )
        wi = jax.lax.broadcasted_iota(jnp.int32, (_GW, W), 1).astype(
            jnp.float32
        )
        ux = jnp.clip((wi + 0.5) / sx - 0.5, 0.0, _GW - 1.0)
        mxT = jnp.maximum(1.0 - jnp.abs(xi - ux), 0.0)
        g = grid_ref[0].reshape(_GH * _C * _GD, _GW)
        gx_s[...] = jnp.dot(
            g, mxT, preferred_element_type=jnp.float32
        ).reshape(_GH, _C * _GD, W)

    for s in range(_NSUB):
        ts = t * _NSUB + s
        r0 = s * _SUB
        hh = (
            jax.lax.broadcasted_iota(jnp.int32, (_SUB, 1), 0).astype(
                jnp.float32
            )
            + ts * _SUB
        )
        uy = jnp.clip((hh + 0.5) / sy - 0.5, 0.0, _GH - 1.0)
        S = jnp.clip((ts - 1) // 2, 0, _GH - 2)
        w1 = jnp.clip(uy - S.astype(jnp.float32), 0.0, 1.0)
        w0 = 1.0 - w1
        rows = slice(r0, r0 + _SUB)

        uz = jnp.clip(guide_ref[0, rows] * _GD - 0.5, 0.0, _GD - 1.0)
        inp = [inp_ref[0, i, rows] for i in range(_CI)]
        acc = [None] * _CO
        for k in range(_GD):
            wk = jnp.maximum(1.0 - jnp.abs(uz - float(k)), 0.0)
            wa = wk * w0
            wb = wk * w1
            for c in range(_CO):
                base = (_NIN * c) * _GD + k
                ta = gx_s[S, base + _CI * _GD]
                tb = gx_s[S + 1, base + _CI * _GD]
                for i in range(_CI):
                    ta = ta + gx_s[S, base + i * _GD] * inp[i]
                    tb = tb + gx_s[S + 1, base + i * _GD] * inp[i]
                term = wa * ta + wb * tb
                acc[c] = term if acc[c] is None else acc[c] + term
        for c in range(_CO):
            out_ref[0, c, rows] = acc[c]


def kernel(bilateral_grid, guide, input):
    N, C, gd, gh, gw = bilateral_grid.shape
    _, ci, H, W = input.shape
    grid_r = jnp.transpose(bilateral_grid, (0, 3, 1, 2, 4)).reshape(
        N, gh, C * gd, gw
    )
    nt = H // _TH
    body = functools.partial(_slice_body, H=H, W=W)
    return pl.pallas_call(
        body,
        grid=(N, nt),
        in_specs=[
            pl.BlockSpec((1, gh, C * gd, gw), lambda n, t: (n, 0, 0, 0)),
            pl.BlockSpec((1, _TH, W), lambda n, t: (n, t, 0)),
            pl.BlockSpec((1, ci, _TH, W), lambda n, t: (n, 0, t, 0)),
        ],
        out_specs=pl.BlockSpec((1, _CO, _TH, W), lambda n, t: (n, 0, t, 0)),
        out_shape=jax.ShapeDtypeStruct((N, _CO, H, W), jnp.float32),
        scratch_shapes=[pltpu.VMEM((gh, C * gd, W), jnp.float32)],
        compiler_params=pltpu.CompilerParams(
            dimension_semantics=("parallel", "arbitrary"),
        ),
    )(grid_r, guide, input)
